# Initial kernel scaffold; baseline (speedup 1.0000x reference)
import jax
import jax.numpy as jnp
from jax.experimental import pallas as pl


def kernel(user_index, item_index, user_embedding, item_embedding):
    raise NotImplementedError("write your pallas kernel here")


# speedup vs baseline: 1.0015x; 1.0015x over previous
import dataclasses
import functools

import jax
import jax.numpy as jnp
from jax import lax
from jax.experimental import pallas as pl
from jax.experimental.pallas import tpu as pltpu
from jax.experimental.pallas import tpu_sc as plsc

NUM_USERS = 100000
NUM_ITEMS = 100000
EMB_DIM = 128
BATCH = 16384

NC, NS, L = 2, 16, 16
NW = NC * NS
B_PER_W = BATCH // NW
CHUNK = 128

_mesh = plsc.VectorSubcoreMesh(core_axis_name="c", subcore_axis_name="s")

_cp = pltpu.CompilerParams()
if "needs_layout_passes" in pltpu.CompilerParams.__dataclass_fields__:
    _cp = dataclasses.replace(_cp, needs_layout_passes=False)


@jax.jit
def _mf_forward(user_index, item_index, user_embedding, item_embedding):
    @functools.partial(
        pl.kernel,
        mesh=_mesh,
        compiler_params=_cp,
        out_type=jax.ShapeDtypeStruct((BATCH,), jnp.float32),
        scratch_types=[
            pltpu.VMEM((B_PER_W,), jnp.int32),
            pltpu.VMEM((B_PER_W,), jnp.int32),
            pltpu.VMEM((CHUNK, EMB_DIM), jnp.float32),
            pltpu.VMEM((CHUNK, EMB_DIM), jnp.float32),
            pltpu.VMEM((B_PER_W,), jnp.float32),
            pltpu.SemaphoreType.DMA,
            pltpu.SemaphoreType.DMA,
        ],
    )
    def k(uidx_hbm, iidx_hbm, utab_hbm, itab_hbm, out_hbm,
          uidx_v, iidx_v, u_v, i_v, o_v, sem_u, sem_i):
        wid = lax.axis_index("s") * NC + lax.axis_index("c")
        base = wid * B_PER_W
        pltpu.sync_copy(uidx_hbm.at[pl.ds(base, B_PER_W)], uidx_v)
        pltpu.sync_copy(iidx_hbm.at[pl.ds(base, B_PER_W)], iidx_v)

        @pl.loop(0, B_PER_W, step=CHUNK)
        def _(c0):
            cu = pltpu.async_copy(
                utab_hbm.at[uidx_v.at[pl.ds(c0, CHUNK)]], u_v, sem_u)
            ci = pltpu.async_copy(
                itab_hbm.at[iidx_v.at[pl.ds(c0, CHUNK)]], i_v, sem_i)
            cu.wait()
            ci.wait()

            lane = lax.iota(jnp.int32, L)

            @pl.loop(0, CHUNK, step=L)
            def _(g):
                ovec = jnp.zeros((L,), jnp.float32)
                for j in range(L):
                    r = g + j
                    acc = u_v[r, pl.ds(0, L)] * i_v[r, pl.ds(0, L)]
                    for sg in range(1, EMB_DIM // L):
                        acc = acc + (u_v[r, pl.ds(sg * L, L)]
                                     * i_v[r, pl.ds(sg * L, L)])
                    ovec = jnp.where(lane == j, jnp.sum(acc), ovec)
                o_v[pl.ds(c0 + g, L)] = ovec

        pltpu.sync_copy(o_v, out_hbm.at[pl.ds(base, B_PER_W)])

    return k(user_index, item_index, user_embedding, item_embedding)


def kernel(user_index, item_index, user_embedding, item_embedding):
    return _mf_forward(user_index.astype(jnp.int32),
                       item_index.astype(jnp.int32),
                       user_embedding, item_embedding)


# speedup vs baseline: 1.2750x; 1.2731x over previous
import dataclasses
import functools

import jax
import jax.numpy as jnp
from jax import lax
from jax.experimental import pallas as pl
from jax.experimental.pallas import tpu as pltpu
from jax.experimental.pallas import tpu_sc as plsc

NUM_USERS = 100000
NUM_ITEMS = 100000
EMB_DIM = 128
BATCH = 16384

NC, NS, L = 2, 16, 16
NW = NC * NS
B_PER_W = BATCH // NW
CHUNK = 128
NCHUNK = B_PER_W // CHUNK

_mesh = plsc.VectorSubcoreMesh(core_axis_name="c", subcore_axis_name="s")

_cp = pltpu.CompilerParams()
if "needs_layout_passes" in pltpu.CompilerParams.__dataclass_fields__:
    _cp = dataclasses.replace(_cp, needs_layout_passes=False)


@jax.jit
def _mf_forward(user_index, item_index, user_embedding, item_embedding):
    @functools.partial(
        pl.kernel,
        mesh=_mesh,
        compiler_params=_cp,
        out_type=jax.ShapeDtypeStruct((BATCH,), jnp.float32),
        scratch_types=[
            pltpu.VMEM((B_PER_W,), jnp.int32),
            pltpu.VMEM((B_PER_W,), jnp.int32),
            pltpu.VMEM((CHUNK, EMB_DIM), jnp.float32),
            pltpu.VMEM((CHUNK, EMB_DIM), jnp.float32),
            pltpu.VMEM((CHUNK, EMB_DIM), jnp.float32),
            pltpu.VMEM((CHUNK, EMB_DIM), jnp.float32),
            pltpu.VMEM((L, L), jnp.float32),
            pltpu.VMEM((B_PER_W,), jnp.float32),
            pltpu.SemaphoreType.DMA,
            pltpu.SemaphoreType.DMA,
        ],
    )
    def k(uidx_hbm, iidx_hbm, utab_hbm, itab_hbm, out_hbm,
          uidx_v, iidx_v, u0_v, i0_v, u1_v, i1_v, acc_v, o_v, sem0, sem1):
        wid = lax.axis_index("s") * NC + lax.axis_index("c")
        base = wid * B_PER_W
        pltpu.sync_copy(uidx_hbm.at[pl.ds(base, B_PER_W)], uidx_v)
        pltpu.sync_copy(iidx_hbm.at[pl.ds(base, B_PER_W)], iidx_v)

        bufs = ((u0_v, i0_v, sem0), (u1_v, i1_v, sem1))

        def issue(g):
            u_b, i_b, sem = bufs[g % 2]
            cu = pltpu.async_copy(
                utab_hbm.at[uidx_v.at[pl.ds(g * CHUNK, CHUNK)]], u_b, sem)
            ci = pltpu.async_copy(
                itab_hbm.at[iidx_v.at[pl.ds(g * CHUNK, CHUNK)]], i_b, sem)
            return cu, ci

        rows = lax.iota(jnp.int32, L)

        def compute(g):
            u_b, i_b, _ = bufs[g % 2]

            @pl.loop(0, CHUNK, step=L)
            def _(r0):
                for j in range(L):
                    r = r0 + j
                    acc = u_b[r, pl.ds(0, L)] * i_b[r, pl.ds(0, L)]
                    for sg in range(1, EMB_DIM // L):
                        acc = acc + (u_b[r, pl.ds(sg * L, L)]
                                     * i_b[r, pl.ds(sg * L, L)])
                    acc_v[j] = acc
                tot = plsc.load_gather(acc_v, [rows, jnp.full((L,), 0, jnp.int32)])
                for col in range(1, L):
                    tot = tot + plsc.load_gather(
                        acc_v, [rows, jnp.full((L,), col, jnp.int32)])
                o_v[pl.ds(g * CHUNK + r0, L)] = tot

        pending = issue(0)
        for g in range(NCHUNK):
            nxt = issue(g + 1) if g + 1 < NCHUNK else None
            pending[0].wait()
            pending[1].wait()
            compute(g)
            pending = nxt

        pltpu.sync_copy(o_v, out_hbm.at[pl.ds(base, B_PER_W)])

    return k(user_index, item_index, user_embedding, item_embedding)


def kernel(user_index, item_index, user_embedding, item_embedding):
    return _mf_forward(user_index.astype(jnp.int32),
                       item_index.astype(jnp.int32),
                       user_embedding, item_embedding)


# speedup vs baseline: 1.3626x; 1.0687x over previous
import dataclasses
import functools

import jax
import jax.numpy as jnp
from jax import lax
from jax.experimental import pallas as pl
from jax.experimental.pallas import tpu as pltpu
from jax.experimental.pallas import tpu_sc as plsc

NUM_USERS = 100000
NUM_ITEMS = 100000
EMB_DIM = 128
BATCH = 16384

NC, NS, L = 2, 16, 16
NW = NC * NS
B_PER_W = BATCH // NW
CHUNK = 128
NCHUNK = B_PER_W // CHUNK

_mesh = plsc.VectorSubcoreMesh(core_axis_name="c", subcore_axis_name="s")

_cp = pltpu.CompilerParams()
if "needs_layout_passes" in pltpu.CompilerParams.__dataclass_fields__:
    _cp = dataclasses.replace(_cp, needs_layout_passes=False)


@jax.jit
def _mf_forward(user_index, item_index, user_embedding, item_embedding):
    @functools.partial(
        pl.kernel,
        mesh=_mesh,
        compiler_params=_cp,
        out_type=jax.ShapeDtypeStruct((BATCH,), jnp.float32),
        scratch_types=[
            pltpu.VMEM((B_PER_W,), jnp.int32),
            pltpu.VMEM((B_PER_W,), jnp.int32),
            pltpu.VMEM((2, CHUNK, EMB_DIM), jnp.float32),
            pltpu.VMEM((2, CHUNK, EMB_DIM), jnp.float32),
            pltpu.VMEM((L, L), jnp.float32),
            pltpu.VMEM((B_PER_W,), jnp.float32),
            pltpu.SemaphoreType.DMA((2,)),
        ],
    )
    def k(uidx_hbm, iidx_hbm, utab_hbm, itab_hbm, out_hbm,
          uidx_v, iidx_v, u_v, i_v, acc_v, o_v, semr):
        wid = lax.axis_index("s") * NC + lax.axis_index("c")
        base = wid * B_PER_W
        pltpu.sync_copy(uidx_hbm.at[pl.ds(base, B_PER_W)], uidx_v)
        pltpu.sync_copy(iidx_hbm.at[pl.ds(base, B_PER_W)], iidx_v)

        def issue(g, sel):
            pltpu.async_copy(
                utab_hbm.at[uidx_v.at[pl.ds(g * CHUNK, CHUNK)]],
                u_v.at[sel], semr.at[sel])
            pltpu.async_copy(
                itab_hbm.at[iidx_v.at[pl.ds(g * CHUNK, CHUNK)]],
                i_v.at[sel], semr.at[sel])

        def drain(g, sel):
            pltpu.make_async_copy(
                utab_hbm.at[uidx_v.at[pl.ds(g * CHUNK, CHUNK)]],
                u_v.at[sel], semr.at[sel]).wait()
            pltpu.make_async_copy(
                itab_hbm.at[iidx_v.at[pl.ds(g * CHUNK, CHUNK)]],
                i_v.at[sel], semr.at[sel]).wait()

        rows = lax.iota(jnp.int32, L)

        issue(0, 0)

        @pl.loop(0, NCHUNK)
        def _(g):
            sel = lax.rem(g, 2)
            nsel = 1 - sel

            @pl.when(g + 1 < NCHUNK)
            def _():
                issue(g + 1, nsel)

            drain(g, sel)

            @pl.loop(0, CHUNK, step=L)
            def _(r0):
                for j in range(L):
                    r = r0 + j
                    acc = (u_v[sel, r, pl.ds(0, L)]
                           * i_v[sel, r, pl.ds(0, L)])
                    for sg in range(1, EMB_DIM // L):
                        acc = acc + (u_v[sel, r, pl.ds(sg * L, L)]
                                     * i_v[sel, r, pl.ds(sg * L, L)])
                    acc_v[j] = acc
                tot = plsc.load_gather(
                    acc_v, [rows, jnp.full((L,), 0, jnp.int32)])
                for col in range(1, L):
                    tot = tot + plsc.load_gather(
                        acc_v, [rows, jnp.full((L,), col, jnp.int32)])
                o_v[pl.ds(g * CHUNK + r0, L)] = tot

        pltpu.sync_copy(o_v, out_hbm.at[pl.ds(base, B_PER_W)])

    return k(user_index, item_index, user_embedding, item_embedding)


def kernel(user_index, item_index, user_embedding, item_embedding):
    return _mf_forward(user_index.astype(jnp.int32),
                       item_index.astype(jnp.int32),
                       user_embedding, item_embedding)


# speedup vs baseline: 1.3629x; 1.0003x over previous
import dataclasses
import functools

import jax
import jax.numpy as jnp
from jax import lax
from jax.experimental import pallas as pl
from jax.experimental.pallas import tpu as pltpu
from jax.experimental.pallas import tpu_sc as plsc

NUM_USERS = 100000
NUM_ITEMS = 100000
EMB_DIM = 128
BATCH = 16384

NC, NS, L = 2, 16, 16
NW = NC * NS
B_PER_W = BATCH // NW
CHUNK = 128
NCHUNK = B_PER_W // CHUNK

_mesh = plsc.VectorSubcoreMesh(core_axis_name="c", subcore_axis_name="s")

_cp = pltpu.CompilerParams()
if "needs_layout_passes" in pltpu.CompilerParams.__dataclass_fields__:
    _cp = dataclasses.replace(_cp, needs_layout_passes=False)


@jax.jit
def _mf_forward(user_index, item_index, user_embedding, item_embedding):
    @functools.partial(
        pl.kernel,
        mesh=_mesh,
        compiler_params=_cp,
        out_type=jax.ShapeDtypeStruct((BATCH,), jnp.float32),
        scratch_types=[
            pltpu.VMEM((B_PER_W,), jnp.int32),
            pltpu.VMEM((B_PER_W,), jnp.int32),
            pltpu.VMEM((2, CHUNK, EMB_DIM), jnp.float32),
            pltpu.VMEM((2, CHUNK, EMB_DIM), jnp.float32),
            pltpu.VMEM((L, L + 1), jnp.float32),
            pltpu.VMEM((B_PER_W,), jnp.float32),
            pltpu.SemaphoreType.DMA((2,)),
        ],
    )
    def k(uidx_hbm, iidx_hbm, utab_hbm, itab_hbm, out_hbm,
          uidx_v, iidx_v, u_v, i_v, acc_v, o_v, semr):
        wid = lax.axis_index("s") * NC + lax.axis_index("c")
        base = wid * B_PER_W
        pltpu.sync_copy(uidx_hbm.at[pl.ds(base, B_PER_W)], uidx_v)
        pltpu.sync_copy(iidx_hbm.at[pl.ds(base, B_PER_W)], iidx_v)

        def issue(g, sel):
            pltpu.async_copy(
                utab_hbm.at[uidx_v.at[pl.ds(g * CHUNK, CHUNK)]],
                u_v.at[sel], semr.at[sel])
            pltpu.async_copy(
                itab_hbm.at[iidx_v.at[pl.ds(g * CHUNK, CHUNK)]],
                i_v.at[sel], semr.at[sel])

        def drain(g, sel):
            pltpu.make_async_copy(
                utab_hbm.at[uidx_v.at[pl.ds(g * CHUNK, CHUNK)]],
                u_v.at[sel], semr.at[sel]).wait()
            pltpu.make_async_copy(
                itab_hbm.at[iidx_v.at[pl.ds(g * CHUNK, CHUNK)]],
                i_v.at[sel], semr.at[sel]).wait()

        rows = lax.iota(jnp.int32, L)

        issue(0, 0)

        @pl.loop(0, NCHUNK)
        def _(g):
            sel = lax.rem(g, 2)
            nsel = 1 - sel

            @pl.when(g + 1 < NCHUNK)
            def _():
                issue(g + 1, nsel)

            drain(g, sel)

            @pl.loop(0, CHUNK, step=L)
            def _(r0):
                for j in range(L):
                    r = r0 + j
                    acc = (u_v[sel, r, pl.ds(0, L)]
                           * i_v[sel, r, pl.ds(0, L)])
                    for sg in range(1, EMB_DIM // L):
                        acc = acc + (u_v[sel, r, pl.ds(sg * L, L)]
                                     * i_v[sel, r, pl.ds(sg * L, L)])
                    acc_v[j, pl.ds(0, L)] = acc
                tot = plsc.load_gather(
                    acc_v, [rows, jnp.full((L,), 0, jnp.int32)])
                for col in range(1, L):
                    tot = tot + plsc.load_gather(
                        acc_v, [rows, jnp.full((L,), col, jnp.int32)])
                o_v[pl.ds(g * CHUNK + r0, L)] = tot

        pltpu.sync_copy(o_v, out_hbm.at[pl.ds(base, B_PER_W)])

    return k(user_index, item_index, user_embedding, item_embedding)


def kernel(user_index, item_index, user_embedding, item_embedding):
    return _mf_forward(user_index.astype(jnp.int32),
                       item_index.astype(jnp.int32),
                       user_embedding, item_embedding)
